# Initial kernel scaffold; baseline (speedup 1.0000x reference)
import jax
import jax.numpy as jnp
from jax.experimental import pallas as pl


def kernel(x_nchw, w_oihw):
    raise NotImplementedError("write your pallas kernel here")


# speedup vs baseline: 1.9715x; 1.9715x over previous
import functools

import jax
import jax.numpy as jnp
from jax.experimental import pallas as pl
from jax.experimental.pallas import tpu as pltpu


def _conv3x3_kernel(x_ref, w_ref, o_ref, *, H, W):
    L = H * W
    C_in = x_ref.shape[1]
    x = x_ref[0].astype(jnp.bfloat16)

    zrow = jnp.zeros((C_in, W), jnp.bfloat16)
    x_up = jnp.concatenate([zrow, x[:, : L - W]], axis=1)
    x_dn = jnp.concatenate([x[:, W:], zrow], axis=1)
    p = jnp.concatenate([x_up, x, x_dn], axis=0)

    zcol = jnp.zeros((3 * C_in, 1), jnp.bfloat16)
    p_m = jnp.concatenate([zcol, p[:, : L - 1]], axis=1)
    p_p = jnp.concatenate([p[:, 1:], zcol], axis=1)
    wcol = jax.lax.broadcasted_iota(jnp.int32, (3 * C_in, L), 1) % W
    p_m = jnp.where(wcol == 0, jnp.bfloat16(0), p_m)
    p_p = jnp.where(wcol == W - 1, jnp.bfloat16(0), p_p)

    acc = jnp.dot(w_ref[1], p, preferred_element_type=jnp.float32)
    acc = acc + jnp.dot(w_ref[0], p_m, preferred_element_type=jnp.float32)
    acc = acc + jnp.dot(w_ref[2], p_p, preferred_element_type=jnp.float32)
    o_ref[0] = acc


def kernel(x_nchw, w_oihw):
    N, C_in, H, W = x_nchw.shape
    C_out, C_in_w, KH, KW = w_oihw.shape
    assert C_in == C_in_w and KH == KW == 3
    L = H * W

    x_flat = x_nchw.reshape(N, C_in, L)
    w2 = jnp.transpose(w_oihw, (3, 0, 2, 1)).reshape(KW, C_out, KH * C_in)
    w2 = w2.astype(jnp.bfloat16)

    body = functools.partial(_conv3x3_kernel, H=H, W=W)
    out_flat = pl.pallas_call(
        body,
        out_shape=jax.ShapeDtypeStruct((N, C_out, L), x_nchw.dtype),
        grid_spec=pltpu.PrefetchScalarGridSpec(
            num_scalar_prefetch=0,
            grid=(N,),
            in_specs=[
                pl.BlockSpec((1, C_in, L), lambda n: (n, 0, 0)),
                pl.BlockSpec((KW, C_out, KH * C_in), lambda n: (0, 0, 0)),
            ],
            out_specs=pl.BlockSpec((1, C_out, L), lambda n: (n, 0, 0)),
        ),
        compiler_params=pltpu.CompilerParams(
            dimension_semantics=("parallel",)),
    )(x_flat, w2)
    return out_flat.reshape(N, C_out, H, W)


# speedup vs baseline: 2.1987x; 1.1152x over previous
import functools

import jax
import jax.numpy as jnp
from jax.experimental import pallas as pl
from jax.experimental.pallas import tpu as pltpu


def _conv3x3_kernel(x_ref, w_ref, o_ref, *, H, W):
    L = H * W
    C_in = x_ref.shape[1]
    for b in range(x_ref.shape[0]):
        x = x_ref[b].astype(jnp.bfloat16)

        zrow = jnp.zeros((C_in, W), jnp.bfloat16)
        x_up = jnp.concatenate([zrow, x[:, : L - W]], axis=1)
        x_dn = jnp.concatenate([x[:, W:], zrow], axis=1)
        p = jnp.concatenate([x_up, x, x_dn], axis=0)

        zcol = jnp.zeros((3 * C_in, 1), jnp.bfloat16)
        p_m = jnp.concatenate([zcol, p[:, : L - 1]], axis=1)
        p_p = jnp.concatenate([p[:, 1:], zcol], axis=1)
        wcol = jax.lax.broadcasted_iota(jnp.int32, (3 * C_in, L), 1) % W
        p_m = jnp.where(wcol == 0, jnp.bfloat16(0), p_m)
        p_p = jnp.where(wcol == W - 1, jnp.bfloat16(0), p_p)

        acc = jnp.dot(w_ref[1], p, preferred_element_type=jnp.float32)
        acc = acc + jnp.dot(w_ref[0], p_m, preferred_element_type=jnp.float32)
        acc = acc + jnp.dot(w_ref[2], p_p, preferred_element_type=jnp.float32)
        o_ref[b] = acc


def kernel(x_nchw, w_oihw):
    N, C_in, H, W = x_nchw.shape
    C_out, C_in_w, KH, KW = w_oihw.shape
    assert C_in == C_in_w and KH == KW == 3
    L = H * W

    x_flat = x_nchw.reshape(N, C_in, L)
    w2 = jnp.transpose(w_oihw, (3, 0, 2, 1)).reshape(KW, C_out, KH * C_in)
    w2 = w2.astype(jnp.bfloat16)

    B = 4 if N % 4 == 0 else 1
    body = functools.partial(_conv3x3_kernel, H=H, W=W)
    out_flat = pl.pallas_call(
        body,
        out_shape=jax.ShapeDtypeStruct((N, C_out, L), x_nchw.dtype),
        grid_spec=pltpu.PrefetchScalarGridSpec(
            num_scalar_prefetch=0,
            grid=(N // B,),
            in_specs=[
                pl.BlockSpec((B, C_in, L), lambda n: (n, 0, 0)),
                pl.BlockSpec((KW, C_out, KH * C_in), lambda n: (0, 0, 0)),
            ],
            out_specs=pl.BlockSpec((B, C_out, L), lambda n: (n, 0, 0)),
        ),
        compiler_params=pltpu.CompilerParams(
            dimension_semantics=("parallel",)),
    )(x_flat, w2)
    return out_flat.reshape(N, C_out, H, W)
